# baseline (speedup 1.0000x reference)
import functools

import jax
import jax.numpy as jnp
from jax import lax
from jax.experimental import pallas as pl
from jax.experimental.pallas import tpu as pltpu
from jax.experimental.pallas import tpu_sc as plsc

_LANES = 16
_NBUF = 4


@functools.lru_cache(maxsize=None)
def _emb_call(B, S, V, D):
    info = plsc.get_sparse_core_info()
    NC, NS = info.num_cores, info.num_subcores
    NW = NC * NS
    assert S % NW == 0
    s_per_w = S // NW
    SP = min(16, s_per_w)
    HP = SP // 2
    assert s_per_w % SP == 0 and D % _LANES == 0 and HP % 8 == 0
    n_chunks = s_per_w // SP
    NU = n_chunks * B
    mesh = plsc.VectorSubcoreMesh(core_axis_name="c", subcore_axis_name="s")

    @functools.partial(
        pl.kernel,
        mesh=mesh,
        out_type=jax.ShapeDtypeStruct((B * S, D), jnp.float32),
        scratch_types=[
            pltpu.VMEM((B * s_per_w,), jnp.int32),
        ] + [pltpu.VMEM((SP, D), jnp.float32) for _ in range(_NBUF + 2)] + [
            pltpu.SemaphoreType.DMA for _ in range(2 * _NBUF + 3)
        ],
    )
    def emb(x_hbm, tok_hbm, pos_hbm, out_hbm, idx_all, *rest):
        toks = list(rest[:_NBUF])
        poss = list(rest[_NBUF:_NBUF + 2])
        sgs = list(rest[_NBUF + 2:2 * _NBUF + 2])
        sss = list(rest[2 * _NBUF + 2:3 * _NBUF + 2])
        sps = list(rest[3 * _NBUF + 2:3 * _NBUF + 4])
        si = rest[3 * _NBUF + 4]
        wid = lax.axis_index("s") * NC + lax.axis_index("c")
        s0 = wid * s_per_w
        units = [(ci, b) for ci in range(n_chunks) for b in range(B)]

        idx_descs = [
            pltpu.async_copy(x_hbm.at[pl.ds(b * S + s0, s_per_w)],
                             idx_all.at[pl.ds(b * s_per_w, s_per_w)], si)
            for b in range(B)
        ]
        for d in idx_descs:
            d.wait()

        def start_gather_half(u, h):
            ci, b = units[u]
            r0 = h * HP
            idx_ref = idx_all.at[pl.ds(b * s_per_w + ci * SP + r0, HP)]
            return pltpu.async_copy(tok_hbm.at[idx_ref],
                                    toks[u % _NBUF].at[pl.ds(r0, HP)],
                                    sgs[u % _NBUF])

        def start_pos(ci):
            return pltpu.async_copy(pos_hbm.at[pl.ds(s0 + ci * SP, SP)],
                                    poss[ci % 2], sps[ci % 2])

        CHUNK = D // _LANES // 4

        def add_half(u, h):
            ci, b = units[u]
            tok_v, pos_v = toks[u % _NBUF], poss[ci % 2]

            def body(t, _):
                r = h * HP + (t >> 2)
                c0 = (t & 3) * (CHUNK * _LANES)
                for c in range(CHUNK):
                    sl = pl.ds(c0 + c * _LANES, _LANES)
                    tok_v[r, sl] = tok_v[r, sl] + pos_v[r, sl]
                return 0

            lax.fori_loop(0, HP * 4, body, 0)

        pos_descs = {0: start_pos(0)}
        g_descs = {(0, 0): start_gather_half(0, 0),
                   (0, 1): start_gather_half(0, 1)}
        s_descs = {}
        for u in range(NU):
            ci, b = units[u]
            slot = u % _NBUF
            if b == 0 and ci + 1 < n_chunks:
                pos_descs[ci + 1] = start_pos(ci + 1)
            if u + 2 - _NBUF in s_descs:
                s_descs.pop(u + 2 - _NBUF).wait()
            g_descs.pop((u, 0)).wait()
            g_descs.pop((u, 1)).wait()
            if b == 0:
                pos_descs.pop(ci).wait()
            if u + 1 < NU:
                g_descs[(u + 1, 0)] = start_gather_half(u + 1, 0)
            add_half(u, 0)
            if u + 1 < NU:
                g_descs[(u + 1, 1)] = start_gather_half(u + 1, 1)
            add_half(u, 1)
            s_descs[u] = pltpu.async_copy(
                toks[slot], out_hbm.at[pl.ds(b * S + s0 + ci * SP, SP)],
                sss[slot])
        for u in sorted(s_descs):
            s_descs.pop(u).wait()

    return emb


def kernel(x, tok_table, pos_table):
    B, S = x.shape
    V, D = tok_table.shape
    x_flat = x.reshape(B * S).astype(jnp.int32)
    out = _emb_call(B, S, V, D)(x_flat, tok_table, pos_table)
    return out.reshape(B, S, D)


# speedup vs baseline: 1.3215x; 1.3215x over previous
import functools

import jax
import jax.numpy as jnp
from jax import lax
from jax.experimental import pallas as pl
from jax.experimental.pallas import tpu as pltpu
from jax.experimental.pallas import tpu_sc as plsc

_LANES = 16
_NBUF = 3


@functools.lru_cache(maxsize=None)
def _emb_call(B, S, V, D):
    info = plsc.get_sparse_core_info()
    NC, NS = info.num_cores, info.num_subcores
    NW = NC * NS
    assert S % NW == 0
    s_per_w = S // NW
    SP = min(16, s_per_w)
    assert s_per_w % SP == 0 and D % _LANES == 0
    n_chunks = s_per_w // SP
    NU = n_chunks * B
    mesh = plsc.VectorSubcoreMesh(core_axis_name="c", subcore_axis_name="s")

    @functools.partial(
        pl.kernel,
        mesh=mesh,
        out_type=jax.ShapeDtypeStruct((B * S, D), jnp.float32),
        scratch_types=[
            pltpu.VMEM((B * s_per_w,), jnp.int32),
        ] + [pltpu.VMEM((SP, D), jnp.float32) for _ in range(_NBUF + 2)] + [
            pltpu.SemaphoreType.DMA for _ in range(2 * _NBUF + 3)
        ],
    )
    def emb(x_hbm, tok_hbm, pos_hbm, out_hbm, idx_all, *rest):
        toks = list(rest[:_NBUF])
        poss = list(rest[_NBUF:_NBUF + 2])
        sgs = list(rest[_NBUF + 2:2 * _NBUF + 2])
        sss = list(rest[2 * _NBUF + 2:3 * _NBUF + 2])
        sps = list(rest[3 * _NBUF + 2:3 * _NBUF + 4])
        si = rest[3 * _NBUF + 4]
        wid = lax.axis_index("s") * NC + lax.axis_index("c")
        s0 = wid * s_per_w
        units = [(ci, b) for ci in range(n_chunks) for b in range(B)]

        idx_descs = [
            pltpu.async_copy(x_hbm.at[pl.ds(b * S + s0, s_per_w)],
                             idx_all.at[pl.ds(b * s_per_w, s_per_w)], si)
            for b in range(B)
        ]
        for d in idx_descs:
            d.wait()

        def start_gather(u):
            ci, b = units[u]
            idx_ref = idx_all.at[pl.ds(b * s_per_w + ci * SP, SP)]
            return pltpu.async_copy(tok_hbm.at[idx_ref], toks[u % _NBUF],
                                    sgs[u % _NBUF])

        def start_pos(ci):
            return pltpu.async_copy(pos_hbm.at[pl.ds(s0 + ci * SP, SP)],
                                    poss[ci % 2], sps[ci % 2])

        pos_descs = {0: start_pos(0)}
        g_descs = {0: start_gather(0)}
        s_descs = {}
        for u in range(NU):
            ci, b = units[u]
            slot = u % _NBUF
            if b == 0 and ci + 1 < n_chunks:
                pos_descs[ci + 1] = start_pos(ci + 1)
            if u + 1 < NU:
                if u + 1 - _NBUF in s_descs:
                    s_descs.pop(u + 1 - _NBUF).wait()
                g_descs[u + 1] = start_gather(u + 1)
            g_descs.pop(u).wait()
            if b == 0:
                pos_descs.pop(ci).wait()

            tok_v, pos_v = toks[slot], poss[ci % 2]

            def row_body(r, _):
                for c in range(D // _LANES):
                    sl = pl.ds(c * _LANES, _LANES)
                    plsc.addupdate(tok_v.at[r, sl], pos_v[r, sl])
                return 0

            lax.fori_loop(0, SP, row_body, 0)
            s_descs[u] = pltpu.async_copy(
                tok_v, out_hbm.at[pl.ds(b * S + s0 + ci * SP, SP)],
                sss[slot])
        for u in sorted(s_descs):
            s_descs.pop(u).wait()

    return emb


def kernel(x, tok_table, pos_table):
    B, S = x.shape
    V, D = tok_table.shape
    x_flat = x.reshape(B * S).astype(jnp.int32)
    out = _emb_call(B, S, V, D)(x_flat, tok_table, pos_table)
    return out.reshape(B, S, D)


# speedup vs baseline: 1.3952x; 1.0558x over previous
import functools

import jax
import jax.numpy as jnp
from jax import lax
from jax.experimental import pallas as pl
from jax.experimental.pallas import tpu as pltpu
from jax.experimental.pallas import tpu_sc as plsc

_LANES = 16
_NBUF = 5


@functools.lru_cache(maxsize=None)
def _emb_call(B, S, V, D):
    info = plsc.get_sparse_core_info()
    NC, NS = info.num_cores, info.num_subcores
    NW = NC * NS
    assert S % NW == 0
    s_per_w = S // NW
    SP = min(16, s_per_w)
    assert s_per_w % SP == 0 and D % _LANES == 0
    n_chunks = s_per_w // SP
    NU = n_chunks * B
    mesh = plsc.VectorSubcoreMesh(core_axis_name="c", subcore_axis_name="s")

    @functools.partial(
        pl.kernel,
        mesh=mesh,
        out_type=jax.ShapeDtypeStruct((B * S, D), jnp.float32),
        scratch_types=[
            pltpu.VMEM((B * s_per_w,), jnp.int32),
        ] + [pltpu.VMEM((SP, D), jnp.float32) for _ in range(_NBUF + 2)] + [
            pltpu.SemaphoreType.DMA for _ in range(2 * _NBUF + 3)
        ],
    )
    def emb(x_hbm, tok_hbm, pos_hbm, out_hbm, idx_all, *rest):
        toks = list(rest[:_NBUF])
        poss = list(rest[_NBUF:_NBUF + 2])
        sgs = list(rest[_NBUF + 2:2 * _NBUF + 2])
        sss = list(rest[2 * _NBUF + 2:3 * _NBUF + 2])
        sps = list(rest[3 * _NBUF + 2:3 * _NBUF + 4])
        si = rest[3 * _NBUF + 4]
        wid = lax.axis_index("s") * NC + lax.axis_index("c")
        s0 = wid * s_per_w
        units = [(ci, b) for ci in range(n_chunks) for b in range(B)]

        idx_descs = [
            pltpu.async_copy(x_hbm.at[pl.ds(b * S + s0, s_per_w)],
                             idx_all.at[pl.ds(b * s_per_w, s_per_w)], si)
            for b in range(B)
        ]
        for d in idx_descs:
            d.wait()

        def start_gather(u):
            ci, b = units[u]
            idx_ref = idx_all.at[pl.ds(b * s_per_w + ci * SP, SP)]
            return pltpu.async_copy(tok_hbm.at[idx_ref], toks[u % _NBUF],
                                    sgs[u % _NBUF])

        def start_pos(ci):
            return pltpu.async_copy(pos_hbm.at[pl.ds(s0 + ci * SP, SP)],
                                    poss[ci % 2], sps[ci % 2])

        pos_descs = {0: start_pos(0)}
        g_descs = {0: start_gather(0), 1: start_gather(1)}
        s_descs = {}
        for u in range(NU):
            ci, b = units[u]
            slot = u % _NBUF
            if b == 0 and ci + 1 < n_chunks:
                pos_descs[ci + 1] = start_pos(ci + 1)
            if u + 2 < NU:
                if u + 2 - _NBUF in s_descs:
                    s_descs.pop(u + 2 - _NBUF).wait()
                g_descs[u + 2] = start_gather(u + 2)
            g_descs.pop(u).wait()
            if b == 0:
                pos_descs.pop(ci).wait()

            tok_v, pos_v = toks[slot], poss[ci % 2]

            def row_body(r, _):
                for c in range(D // _LANES):
                    sl = pl.ds(c * _LANES, _LANES)
                    plsc.addupdate(tok_v.at[r, sl], pos_v[r, sl])
                return 0

            lax.fori_loop(0, SP, row_body, 0)
            s_descs[u] = pltpu.async_copy(
                tok_v, out_hbm.at[pl.ds(b * S + s0 + ci * SP, SP)],
                sss[slot])
        for u in sorted(s_descs):
            s_descs.pop(u).wait()

    return emb


def kernel(x, tok_table, pos_table):
    B, S = x.shape
    V, D = tok_table.shape
    x_flat = x.reshape(B * S).astype(jnp.int32)
    out = _emb_call(B, S, V, D)(x_flat, tok_table, pos_table)
    return out.reshape(B, S, D)
